# Initial kernel scaffold; baseline (speedup 1.0000x reference)
import jax
import jax.numpy as jnp
from jax.experimental import pallas as pl


def kernel(x, lut):
    raise NotImplementedError("write your pallas kernel here")


# speedup vs baseline: 13.9342x; 13.9342x over previous
import functools
import math

import jax
import jax.numpy as jnp
import numpy as np
from jax import lax
from jax.experimental import pallas as pl
from jax.experimental.pallas import tpu as pltpu
from jax.experimental.pallas import tpu_sc as plsc

D_MODEL = 1024
SEQ = 4096
NB = 4
ROWS = NB * SEQ
SCALE = math.sqrt(D_MODEL)

L = 16
NC, NS = 2, 16
NW = NC * NS
P_PER_W = SEQ // NW
CP = 32
NCHUNK = P_PER_W // CP


def _make_pe(seq_len: int, d_model: int) -> np.ndarray:
    position = np.arange(seq_len, dtype=np.float32)[:, None]
    div_term = np.exp(
        np.arange(0, d_model, 2, dtype=np.float32) * (-math.log(10000.0) / d_model)
    )
    pe = np.zeros((seq_len, d_model), dtype=np.float32)
    pe[:, 0::2] = np.sin(position * div_term)
    pe[:, 1::2] = np.cos(position * div_term)
    return pe


_PE = _make_pe(SEQ, D_MODEL)


def _sc_embed(x, lut, pe):
    mesh = plsc.VectorSubcoreMesh(core_axis_name="c", subcore_axis_name="s")

    @functools.partial(
        pl.kernel,
        mesh=mesh,
        out_type=jax.ShapeDtypeStruct((ROWS, D_MODEL), jnp.float32),
        scratch_types=[
            pltpu.VMEM((NB, P_PER_W), jnp.int32),
            pltpu.VMEM((CP, D_MODEL), jnp.float32),
            pltpu.VMEM((CP, D_MODEL), jnp.float32),
            pltpu.SemaphoreType.DMA,
        ],
    )
    def k(x_hbm, lut_hbm, pe_hbm, out_hbm, idx_v, pe_v, row_v, sem):
        c = lax.axis_index("c")
        s = lax.axis_index("s")
        wid = s * NC + c
        pbase = wid * P_PER_W
        for b in range(NB):
            pltpu.sync_copy(x_hbm.at[b, pl.ds(pbase, P_PER_W)], idx_v.at[b])

        def chunk(ci, carry):
            off = ci * CP
            pltpu.sync_copy(pe_hbm.at[pl.ds(pbase + off, CP)], pe_v)
            for b in range(NB):
                pltpu.async_copy(
                    lut_hbm.at[idx_v.at[b, pl.ds(off, CP)]], row_v, sem
                ).wait()

                def rowloop(r, cr):
                    for j in range(D_MODEL // L):
                        sl = pl.ds(j * L, L)
                        row_v[r, sl] = row_v[r, sl] * SCALE + pe_v[r, sl]
                    return cr

                lax.fori_loop(0, CP, rowloop, 0)
                pltpu.sync_copy(
                    row_v, out_hbm.at[pl.ds(b * SEQ + pbase + off, CP)]
                )
            return carry

        lax.fori_loop(0, NCHUNK, chunk, 0)

    return k(x, lut, pe)


def kernel(x, lut):
    pe = jnp.asarray(_PE)
    out = _sc_embed(x.astype(jnp.int32), lut, pe)
    return out.reshape(NB, SEQ, D_MODEL)


# speedup vs baseline: 17.8558x; 1.2814x over previous
import functools
import math

import jax
import jax.numpy as jnp
import numpy as np
from jax import lax
from jax.experimental import pallas as pl
from jax.experimental.pallas import tpu as pltpu
from jax.experimental.pallas import tpu_sc as plsc

D_MODEL = 1024
SEQ = 4096
NB = 4
ROWS = NB * SEQ
SCALE = math.sqrt(D_MODEL)

L = 16
NC, NS = 2, 16
NW = NC * NS
P_PER_W = SEQ // NW
CP = 32
NCHUNK = P_PER_W // CP


def _make_pe(seq_len: int, d_model: int) -> np.ndarray:
    position = np.arange(seq_len, dtype=np.float32)[:, None]
    div_term = np.exp(
        np.arange(0, d_model, 2, dtype=np.float32) * (-math.log(10000.0) / d_model)
    )
    pe = np.zeros((seq_len, d_model), dtype=np.float32)
    pe[:, 0::2] = np.sin(position * div_term)
    pe[:, 1::2] = np.cos(position * div_term)
    return pe


_PE = _make_pe(SEQ, D_MODEL)


def _sc_embed(x, lut, pe):
    mesh = plsc.VectorSubcoreMesh(core_axis_name="c", subcore_axis_name="s")

    @functools.partial(
        pl.kernel,
        mesh=mesh,
        out_type=jax.ShapeDtypeStruct((ROWS, D_MODEL), jnp.float32),
        scratch_types=[
            pltpu.VMEM((NB, P_PER_W), jnp.int32),
            pltpu.VMEM((CP, D_MODEL), jnp.float32),
            pltpu.VMEM((CP, D_MODEL), jnp.float32),
            pltpu.VMEM((CP, D_MODEL), jnp.float32),
            pltpu.SemaphoreType.DMA,
            pltpu.SemaphoreType.DMA,
            pltpu.SemaphoreType.DMA,
            pltpu.SemaphoreType.DMA,
        ],
    )
    def k(x_hbm, lut_hbm, pe_hbm, out_hbm, idx_v, pe_v, row0, row1,
          g0, g1, s0, s1):
        c = lax.axis_index("c")
        s = lax.axis_index("s")
        wid = s * NC + c
        pbase = wid * P_PER_W
        rows = (row0, row1)
        gsems = (g0, g1)
        ssems = (s0, s1)
        for b in range(NB):
            pltpu.sync_copy(x_hbm.at[b, pl.ds(pbase, P_PER_W)], idx_v.at[b])

        def gather(b, off, buf, gsem):
            return pltpu.make_async_copy(
                lut_hbm.at[idx_v.at[b, pl.ds(off, CP)]], buf, gsem
            )

        def store(b, off, buf, ssem):
            return pltpu.make_async_copy(
                buf, out_hbm.at[pl.ds(b * SEQ + pbase + off, CP)], ssem
            )

        def chunk(ci, carry):
            off = ci * CP
            gather(0, off, rows[0], gsems[0]).start()
            pltpu.sync_copy(pe_hbm.at[pl.ds(pbase + off, CP)], pe_v)
            for b in range(NB):
                i = b % 2
                if b + 1 < NB:
                    j = (b + 1) % 2
                    if b >= 1:
                        store(b - 1, off, rows[j], ssems[j]).wait()
                    gather(b + 1, off, rows[j], gsems[j]).start()
                gather(b, off, rows[i], gsems[i]).wait()

                def rowloop(r, cr):
                    for v in range(D_MODEL // L):
                        sl = pl.ds(v * L, L)
                        rows_i = rows[i]
                        rows_i[r, sl] = rows_i[r, sl] * SCALE + pe_v[r, sl]
                    return cr

                lax.fori_loop(0, CP, rowloop, 0)
                store(b, off, rows[i], ssems[i]).start()
            store(NB - 2, off, rows[(NB - 2) % 2], ssems[(NB - 2) % 2]).wait()
            store(NB - 1, off, rows[(NB - 1) % 2], ssems[(NB - 1) % 2]).wait()
            return carry

        lax.fori_loop(0, NCHUNK, chunk, 0)

    return k(x, lut, pe)


def kernel(x, lut):
    pe = jnp.asarray(_PE)
    out = _sc_embed(x.astype(jnp.int32), lut, pe)
    return out.reshape(NB, SEQ, D_MODEL)
